# Initial kernel scaffold; baseline (speedup 1.0000x reference)
import functools

import jax
import jax.numpy as jnp
from jax import lax
from jax.experimental import pallas as pl
from jax.experimental.pallas import tpu as pltpu
from jax.experimental.pallas import tpu_sc as plsc

NC = 2
NS = 16
NW = NC * NS
LANE = 128
KGRP = 8

STRIPE = 6256
ACC_ROWS = STRIPE * NS


def _mesh():
    return plsc.VectorSubcoreMesh(core_axis_name="c", subcore_axis_name="s")


def _rows_per_tile(n_edges):
    rows = -(-n_edges // LANE)
    per_tile = -(-rows // (NW * KGRP)) * KGRP
    return per_tile


def _sc_degree(dst_rows, zeros_col, ones_blk, rt):
    def body(dst_hbm, z_hbm, one_hbm, out_hbm, dstv, onev, acc_sh):
        c = lax.axis_index("c")
        s = lax.axis_index("s")
        pltpu.sync_copy(z_hbm.at[pl.ds(s * STRIPE, STRIPE)],
                        acc_sh.at[pl.ds(s * STRIPE, STRIPE)])
        pltpu.sync_copy(one_hbm, onev)
        plsc.subcore_barrier()
        base = (c * NS + s) * rt

        def grp(g, carry):
            r0 = base + g * KGRP
            pltpu.sync_copy(dst_hbm.at[pl.ds(r0, KGRP)], dstv)
            for j in range(KGRP):
                pltpu.sync_copy(onev, acc_sh.at[dstv.at[j]], add=True)
            return carry

        lax.fori_loop(0, rt // KGRP, grp, 0)
        plsc.subcore_barrier()
        pltpu.sync_copy(acc_sh.at[pl.ds(s * STRIPE, STRIPE)],
                        out_hbm.at[pl.ds(c * ACC_ROWS + s * STRIPE, STRIPE)])

    f = pl.kernel(
        body,
        out_type=jax.ShapeDtypeStruct((2 * ACC_ROWS, 1), jnp.float32),
        mesh=_mesh(),
        scratch_types=[
            pltpu.VMEM((KGRP, LANE), jnp.int32),
            pltpu.VMEM((LANE, 1), jnp.float32),
            pltpu.VMEM_SHARED((ACC_ROWS, 1), jnp.float32),
        ],
    )
    return f(dst_rows, zeros_col, ones_blk)


def _sc_aggregate(src_rows, dst_rows, p, zeros_f, rt, feat):
    def body(src_hbm, dst_hbm, p_hbm, z_hbm, out_hbm, srcv, dstv, rowsv,
             acc_sh, sem):
        c = lax.axis_index("c")
        s = lax.axis_index("s")
        pltpu.sync_copy(z_hbm.at[pl.ds(s * STRIPE, STRIPE)],
                        acc_sh.at[pl.ds(s * STRIPE, STRIPE)])
        plsc.subcore_barrier()
        base = (c * NS + s) * rt

        def grp(g, carry):
            r0 = base + g * KGRP
            pltpu.sync_copy(src_hbm.at[pl.ds(r0, KGRP)], srcv)
            pltpu.sync_copy(dst_hbm.at[pl.ds(r0, KGRP)], dstv)
            cps = [pltpu.async_copy(p_hbm.at[srcv.at[j]], rowsv.at[j], sem)
                   for j in range(KGRP)]
            for cp in cps:
                cp.wait()
            for j in range(KGRP):
                pltpu.sync_copy(rowsv.at[j], acc_sh.at[dstv.at[j]], add=True)
            return carry

        lax.fori_loop(0, rt // KGRP, grp, 0)
        plsc.subcore_barrier()
        pltpu.sync_copy(acc_sh.at[pl.ds(s * STRIPE, STRIPE)],
                        out_hbm.at[pl.ds(c * ACC_ROWS + s * STRIPE, STRIPE)])

    f = pl.kernel(
        body,
        out_type=jax.ShapeDtypeStruct((2 * ACC_ROWS, feat), jnp.float32),
        mesh=_mesh(),
        scratch_types=[
            pltpu.VMEM((KGRP, LANE), jnp.int32),
            pltpu.VMEM((KGRP, LANE), jnp.int32),
            pltpu.VMEM((KGRP, LANE, feat), jnp.float32),
            pltpu.VMEM_SHARED((ACC_ROWS, feat), jnp.float32),
            pltpu.SemaphoreType.DMA,
        ],
    )
    return f(src_rows, dst_rows, p, zeros_f)


_ROWS_BLK = 8192


def _grid(n):
    return (-(-n // _ROWS_BLK),)


def _rows_spec(feat):
    return pl.BlockSpec((_ROWS_BLK, feat), lambda i: (i, 0))


def _full_spec(r, c):
    return pl.BlockSpec((r, c), lambda i: (0, 0))


def _stage_a_body(deg0, deg1, x, w, d_out, p_out):
    deg = deg0[...] + deg1[...] + 1.0
    dd = lax.rsqrt(deg)
    d_out[...] = dd
    p_out[...] = dd * jnp.dot(x[...], w[...],
                              preferred_element_type=jnp.float32)


def _stage_mid_body(a0, a1, p, d, b, w, out):
    dd = d[...]
    h = dd * (a0[...] + a1[...] + p[...]) + b[...]
    out[...] = dd * jnp.dot(jnp.tanh(h), w[...],
                            preferred_element_type=jnp.float32)


def _stage_last_body(a0, a1, p, d, b, out):
    out[...] = d[...] * (a0[...] + a1[...] + p[...]) + b[...]


def _tc_stage_a(deg0, deg1, x, w, n):
    fin = x.shape[1]
    return pl.pallas_call(
        _stage_a_body,
        grid=_grid(n),
        in_specs=[_rows_spec(1), _rows_spec(1), _rows_spec(fin),
                  _full_spec(fin, w.shape[1])],
        out_specs=[_rows_spec(1), _rows_spec(w.shape[1])],
        out_shape=[jax.ShapeDtypeStruct((n, 1), jnp.float32),
                   jax.ShapeDtypeStruct((n, w.shape[1]), jnp.float32)],
    )(deg0, deg1, x, w)


def _tc_stage_mid(a0, a1, p, d, b, w, n):
    feat = p.shape[1]
    fout = w.shape[1]
    return pl.pallas_call(
        _stage_mid_body,
        grid=_grid(n),
        in_specs=[_rows_spec(feat), _rows_spec(feat), _rows_spec(feat),
                  _rows_spec(1), _full_spec(1, feat), _full_spec(feat, fout)],
        out_specs=_rows_spec(fout),
        out_shape=jax.ShapeDtypeStruct((n, fout), jnp.float32),
    )(a0, a1, p, d, b, w)


def _tc_stage_last(a0, a1, p, d, b, n):
    feat = p.shape[1]
    return pl.pallas_call(
        _stage_last_body,
        grid=_grid(n),
        in_specs=[_rows_spec(feat), _rows_spec(feat), _rows_spec(feat),
                  _rows_spec(1), _full_spec(1, feat)],
        out_specs=_rows_spec(feat),
        out_shape=jax.ShapeDtypeStruct((n, feat), jnp.float32),
    )(a0, a1, p, d, b)


def kernel(x, edge_index, W1, b1, W2, b2, W3, b3):
    n = x.shape[0]
    e = edge_index.shape[1]
    rt = _rows_per_tile(e)
    e_pad = rt * NW * LANE

    src = edge_index[0].astype(jnp.int32)
    dst = edge_index[1].astype(jnp.int32)
    pad = e_pad - e
    src_rows = jnp.concatenate(
        [src, jnp.zeros((pad,), jnp.int32)]).reshape(rt * NW, LANE)
    dst_rows = jnp.concatenate(
        [dst, jnp.full((pad,), n, jnp.int32)]).reshape(rt * NW, LANE)

    zeros1 = jnp.zeros((ACC_ROWS, 1), jnp.float32)
    zeros4 = jnp.zeros((ACC_ROWS, W1.shape[1]), jnp.float32)
    zeros_last = jnp.zeros((ACC_ROWS, W3.shape[1]), jnp.float32)
    ones_blk = jnp.ones((LANE, 1), jnp.float32)

    deg = _sc_degree(dst_rows, zeros1, ones_blk, rt)
    d, p1 = _tc_stage_a(deg[:ACC_ROWS], deg[ACC_ROWS:], x, W1, n)

    acc1 = _sc_aggregate(src_rows, dst_rows, p1, zeros4, rt, W1.shape[1])
    p2 = _tc_stage_mid(acc1[:n], acc1[ACC_ROWS:ACC_ROWS + n], p1, d,
                       b1.reshape(1, -1), W2, n)

    acc2 = _sc_aggregate(src_rows, dst_rows, p2, zeros4, rt, W2.shape[1])
    p3 = _tc_stage_mid(acc2[:n], acc2[ACC_ROWS:ACC_ROWS + n], p2, d,
                       b2.reshape(1, -1), W3, n)

    acc3 = _sc_aggregate(src_rows, dst_rows, p3, zeros_last, rt, W3.shape[1])
    out = _tc_stage_last(acc3[:n], acc3[ACC_ROWS:ACC_ROWS + n], p3, d,
                         b3.reshape(1, -1), n)
    return out


# speedup vs baseline: 51.5090x; 51.5090x over previous
import jax
import jax.numpy as jnp
from jax import lax
from jax.experimental import pallas as pl
from jax.experimental.pallas import tpu as pltpu
from jax.experimental.pallas import tpu_sc as plsc

NC = 2
NS = 16
NW = NC * NS
LANE = 128
KGRP = 8
FEAT = 8

STRIPE = 6256
ACC_ROWS = STRIPE * NS


def _mesh():
    return plsc.VectorSubcoreMesh(core_axis_name="c", subcore_axis_name="s")


def _rows_per_tile(n_edges):
    rows = -(-n_edges // LANE)
    per_tile = -(-rows // (NW * KGRP)) * KGRP
    return per_tile


def _sc_degree(dst_rows, zeros_f, ones_blk, rt):
    def body(dst_hbm, z_hbm, one_hbm, out_hbm, dstv, onev, acc_sh):
        c = lax.axis_index("c")
        s = lax.axis_index("s")
        pltpu.sync_copy(z_hbm.at[pl.ds(s * STRIPE, STRIPE)],
                        acc_sh.at[pl.ds(s * STRIPE, STRIPE)])
        pltpu.sync_copy(one_hbm, onev)
        plsc.subcore_barrier()
        base = (c * NS + s) * rt

        def grp(g, carry):
            r0 = base + g * KGRP
            pltpu.sync_copy(dst_hbm.at[pl.ds(r0, KGRP)], dstv)
            for j in range(KGRP):
                pltpu.sync_copy(onev, acc_sh.at[dstv.at[j]], add=True)
            return carry

        lax.fori_loop(0, rt // KGRP, grp, 0)
        plsc.subcore_barrier()
        pltpu.sync_copy(acc_sh.at[pl.ds(s * STRIPE, STRIPE)],
                        out_hbm.at[pl.ds(c * ACC_ROWS + s * STRIPE, STRIPE)])

    f = pl.kernel(
        body,
        out_type=jax.ShapeDtypeStruct((2 * ACC_ROWS, FEAT), jnp.float32),
        mesh=_mesh(),
        scratch_types=[
            pltpu.VMEM((KGRP, LANE), jnp.int32),
            pltpu.VMEM((LANE, FEAT), jnp.float32),
            pltpu.VMEM_SHARED((ACC_ROWS, FEAT), jnp.float32),
        ],
        compiler_params=pltpu.CompilerParams(use_tc_tiling_on_sc=False),
    )
    return f(dst_rows, zeros_f, ones_blk)


def _sc_aggregate(src_rows, dst_rows, p, zeros_f, rt):
    def body(src_hbm, dst_hbm, p_hbm, z_hbm, out_hbm, srcv, dstv, rowsv,
             acc_sh, sem):
        c = lax.axis_index("c")
        s = lax.axis_index("s")
        pltpu.sync_copy(z_hbm.at[pl.ds(s * STRIPE, STRIPE)],
                        acc_sh.at[pl.ds(s * STRIPE, STRIPE)])
        plsc.subcore_barrier()
        base = (c * NS + s) * rt

        def grp(g, carry):
            r0 = base + g * KGRP
            pltpu.sync_copy(src_hbm.at[pl.ds(r0, KGRP)], srcv)
            pltpu.sync_copy(dst_hbm.at[pl.ds(r0, KGRP)], dstv)
            cps = [pltpu.async_copy(p_hbm.at[srcv.at[j]], rowsv.at[j], sem)
                   for j in range(KGRP)]
            for cp in cps:
                cp.wait()
            for j in range(KGRP):
                pltpu.sync_copy(rowsv.at[j], acc_sh.at[dstv.at[j]], add=True)
            return carry

        lax.fori_loop(0, rt // KGRP, grp, 0)
        plsc.subcore_barrier()
        pltpu.sync_copy(acc_sh.at[pl.ds(s * STRIPE, STRIPE)],
                        out_hbm.at[pl.ds(c * ACC_ROWS + s * STRIPE, STRIPE)])

    f = pl.kernel(
        body,
        out_type=jax.ShapeDtypeStruct((2 * ACC_ROWS, FEAT), jnp.float32),
        mesh=_mesh(),
        scratch_types=[
            pltpu.VMEM((KGRP, LANE), jnp.int32),
            pltpu.VMEM((KGRP, LANE), jnp.int32),
            pltpu.VMEM((KGRP, LANE, FEAT), jnp.float32),
            pltpu.VMEM_SHARED((ACC_ROWS, FEAT), jnp.float32),
            pltpu.SemaphoreType.DMA,
        ],
        compiler_params=pltpu.CompilerParams(use_tc_tiling_on_sc=False),
    )
    return f(src_rows, dst_rows, p, zeros_f)


_ROWS_BLK = 8192


def _grid(n):
    return (-(-n // _ROWS_BLK),)


def _rows_spec(feat):
    return pl.BlockSpec((_ROWS_BLK, feat), lambda i: (i, 0))


def _full_spec(r, c):
    return pl.BlockSpec((r, c), lambda i: (0, 0))


def _stage_a_body(deg0, deg1, x, w, d_out, p_out):
    deg = deg0[...][:, :1] + deg1[...][:, :1] + 1.0
    dd = lax.rsqrt(deg)
    d_out[...] = dd
    p_out[...] = dd * jnp.dot(x[...], w[...],
                              preferred_element_type=jnp.float32)


def _stage_mid_body(a0, a1, p, d, b, w, out):
    dd = d[...]
    h = dd * (a0[...] + a1[...] + p[...]) + b[...]
    out[...] = dd * jnp.dot(jnp.tanh(h), w[...],
                            preferred_element_type=jnp.float32)


def _stage_last_body(a0, a1, p, d, b, out):
    res = d[...] * (a0[...] + a1[...] + p[...]) + b[...]
    out[...] = res[:, : out.shape[1]]


def _tc_stage_a(deg0, deg1, x, w, n):
    fin = x.shape[1]
    return pl.pallas_call(
        _stage_a_body,
        grid=_grid(n),
        in_specs=[_rows_spec(FEAT), _rows_spec(FEAT), _rows_spec(fin),
                  _full_spec(fin, FEAT)],
        out_specs=[_rows_spec(1), _rows_spec(FEAT)],
        out_shape=[jax.ShapeDtypeStruct((n, 1), jnp.float32),
                   jax.ShapeDtypeStruct((n, FEAT), jnp.float32)],
    )(deg0, deg1, x, w)


def _tc_stage_mid(a0, a1, p, d, b, w, n):
    return pl.pallas_call(
        _stage_mid_body,
        grid=_grid(n),
        in_specs=[_rows_spec(FEAT), _rows_spec(FEAT), _rows_spec(FEAT),
                  _rows_spec(1), _full_spec(1, FEAT), _full_spec(FEAT, FEAT)],
        out_specs=_rows_spec(FEAT),
        out_shape=jax.ShapeDtypeStruct((n, FEAT), jnp.float32),
    )(a0, a1, p, d, b, w)


def _tc_stage_last(a0, a1, p, d, b, n, fout):
    return pl.pallas_call(
        _stage_last_body,
        grid=_grid(n),
        in_specs=[_rows_spec(FEAT), _rows_spec(FEAT), _rows_spec(FEAT),
                  _rows_spec(1), _full_spec(1, FEAT)],
        out_specs=_rows_spec(fout),
        out_shape=jax.ShapeDtypeStruct((n, fout), jnp.float32),
    )(a0, a1, p, d, b)


def _pad_cols(a, cols):
    return jnp.pad(a, ((0, 0), (0, cols - a.shape[1])))


def kernel(x, edge_index, W1, b1, W2, b2, W3, b3):
    n = x.shape[0]
    e = edge_index.shape[1]
    rt = _rows_per_tile(e)
    e_pad = rt * NW * LANE

    src = edge_index[0].astype(jnp.int32)
    dst = edge_index[1].astype(jnp.int32)
    pad = e_pad - e
    src_rows = jnp.concatenate(
        [src, jnp.zeros((pad,), jnp.int32)]).reshape(rt * NW, LANE)
    dst_rows = jnp.concatenate(
        [dst, jnp.full((pad,), n, jnp.int32)]).reshape(rt * NW, LANE)

    w1p = _pad_cols(W1, FEAT)
    w2p = jnp.pad(W2, ((0, FEAT - W2.shape[0]), (0, FEAT - W2.shape[1])))
    w3p = jnp.pad(W3, ((0, FEAT - W3.shape[0]), (0, FEAT - W3.shape[1])))
    b1p = _pad_cols(b1.reshape(1, -1), FEAT)
    b2p = _pad_cols(b2.reshape(1, -1), FEAT)
    b3p = _pad_cols(b3.reshape(1, -1), FEAT)

    zeros_f = jnp.zeros((ACC_ROWS, FEAT), jnp.float32)
    ones_blk = jnp.zeros((LANE, FEAT), jnp.float32).at[:, 0].set(1.0)

    deg = _sc_degree(dst_rows, zeros_f, ones_blk, rt)
    d, p1 = _tc_stage_a(deg[:ACC_ROWS], deg[ACC_ROWS:], x, w1p, n)

    acc1 = _sc_aggregate(src_rows, dst_rows, p1, zeros_f, rt)
    p2 = _tc_stage_mid(acc1[:n], acc1[ACC_ROWS:ACC_ROWS + n], p1, d, b1p,
                       w2p, n)

    acc2 = _sc_aggregate(src_rows, dst_rows, p2, zeros_f, rt)
    p3 = _tc_stage_mid(acc2[:n], acc2[ACC_ROWS:ACC_ROWS + n], p2, d, b2p,
                       w3p, n)

    acc3 = _sc_aggregate(src_rows, dst_rows, p3, zeros_f, rt)
    out = _tc_stage_last(acc3[:n], acc3[ACC_ROWS:ACC_ROWS + n], p3, d, b3p,
                         n, W3.shape[1])
    return out


# speedup vs baseline: 77.4705x; 1.5040x over previous
import jax
import jax.numpy as jnp
from jax import lax
from jax.experimental import pallas as pl
from jax.experimental.pallas import tpu as pltpu
from jax.experimental.pallas import tpu_sc as plsc

NC = 2
NS = 16
NW = NC * NS
GE = 512
FEAT = 16
PACK = 8

STRIPE = 6256
ACC_ROWS = STRIPE * NS
ACC_PACK = ACC_ROWS // PACK


def _mesh():
    return plsc.VectorSubcoreMesh(core_axis_name="c", subcore_axis_name="s")


def _groups_per_tile(n_edges):
    g = -(-n_edges // (NW * GE))
    return g + (g % 2)


def _acc_out_spec():
    return (jax.ShapeDtypeStruct((ACC_ROWS, FEAT), jnp.float32),
            jax.ShapeDtypeStruct((ACC_ROWS, FEAT), jnp.float32))


def _zero_and_barrier(z_hbm, acc_sh, s):
    pltpu.sync_copy(z_hbm.at[pl.ds(s * STRIPE, STRIPE)],
                    acc_sh.at[pl.ds(s * STRIPE, STRIPE)])


def _copy_out(acc_sh, out0, out1, c, s):
    @pl.when(c == 0)
    def _():
        pltpu.sync_copy(acc_sh.at[pl.ds(s * STRIPE, STRIPE)],
                        out0.at[pl.ds(s * STRIPE, STRIPE)])

    @pl.when(c == 1)
    def _():
        pltpu.sync_copy(acc_sh.at[pl.ds(s * STRIPE, STRIPE)],
                        out1.at[pl.ds(s * STRIPE, STRIPE)])


def _sc_degree(dst_flat, zeros_f, ones_blk, gpt):
    def body(dst_hbm, z_hbm, one_hbm, out0, out1, dstv0, dstv1, onev,
             acc_sh, ss0, ss1):
        c = lax.axis_index("c")
        s = lax.axis_index("s")
        _zero_and_barrier(z_hbm, acc_sh, s)
        pltpu.sync_copy(one_hbm, onev)
        plsc.subcore_barrier()
        base = (c * NS + s) * gpt
        dstv = (dstv0, dstv1)
        ss = (ss0, ss1)

        def steps(g, b):
            @pl.when(g >= 2)
            def _():
                pltpu.make_async_copy(onev, acc_sh.at[dstv[b]], ss[b]).wait()
            pltpu.sync_copy(dst_hbm.at[pl.ds((base + g) * GE, GE)], dstv[b])
            pltpu.async_copy(onev, acc_sh.at[dstv[b]], ss[b], add=True)

        def pair(i, carry):
            steps(2 * i, 0)
            steps(2 * i + 1, 1)
            return carry

        lax.fori_loop(0, gpt // 2, pair, 0)
        pltpu.make_async_copy(onev, acc_sh.at[dstv0], ss0).wait()
        pltpu.make_async_copy(onev, acc_sh.at[dstv1], ss1).wait()
        plsc.subcore_barrier()
        _copy_out(acc_sh, out0, out1, c, s)

    f = pl.kernel(
        body,
        out_type=_acc_out_spec(),
        mesh=_mesh(),
        scratch_types=[
            pltpu.VMEM((GE,), jnp.int32),
            pltpu.VMEM((GE,), jnp.int32),
            pltpu.VMEM((GE, FEAT), jnp.float32),
            pltpu.VMEM_SHARED((ACC_ROWS, FEAT), jnp.float32),
            pltpu.SemaphoreType.DMA,
            pltpu.SemaphoreType.DMA,
        ],
        compiler_params=pltpu.CompilerParams(use_tc_tiling_on_sc=False),
    )
    return f(dst_flat, zeros_f, ones_blk)


def _sc_aggregate(src_flat, dst_flat, p, zeros_f, gpt):
    def body(src_hbm, dst_hbm, p_hbm, z_hbm, out0, out1, srcv0, srcv1,
             dstv0, dstv1, rows0, rows1, acc_sh, gs0, gs1, ss0, ss1):
        c = lax.axis_index("c")
        s = lax.axis_index("s")
        _zero_and_barrier(z_hbm, acc_sh, s)
        plsc.subcore_barrier()
        base = (c * NS + s) * gpt
        srcv = (srcv0, srcv1)
        dstv = (dstv0, dstv1)
        rows = (rows0, rows1)
        gs = (gs0, gs1)
        ss = (ss0, ss1)

        def load(g, b):
            off = (base + g) * GE
            pltpu.sync_copy(src_hbm.at[pl.ds(off, GE)], srcv[b])
            pltpu.sync_copy(dst_hbm.at[pl.ds(off, GE)], dstv[b])

        def steps(g, b):
            nb = 1 - b
            @pl.when(g >= 1)
            def _():
                pltpu.make_async_copy(rows[nb], acc_sh.at[dstv[nb]],
                                      ss[nb]).wait()

            @pl.when(g + 1 < gpt)
            def _():
                load(g + 1, nb)
            pltpu.make_async_copy(p_hbm.at[srcv[b]], rows[b], gs[b]).wait()
            pltpu.async_copy(rows[b], acc_sh.at[dstv[b]], ss[b], add=True)

            @pl.when(g + 1 < gpt)
            def _():
                pltpu.async_copy(p_hbm.at[srcv[nb]], rows[nb], gs[nb])

        def pair(i, carry):
            steps(2 * i, 0)
            steps(2 * i + 1, 1)
            return carry

        load(0, 0)
        pltpu.async_copy(p_hbm.at[srcv0], rows0, gs0)
        lax.fori_loop(0, gpt // 2, pair, 0)
        pltpu.make_async_copy(rows1, acc_sh.at[dstv1], ss1).wait()
        plsc.subcore_barrier()
        _copy_out(acc_sh, out0, out1, c, s)

    f = pl.kernel(
        body,
        out_type=_acc_out_spec(),
        mesh=_mesh(),
        scratch_types=[
            pltpu.VMEM((GE,), jnp.int32),
            pltpu.VMEM((GE,), jnp.int32),
            pltpu.VMEM((GE,), jnp.int32),
            pltpu.VMEM((GE,), jnp.int32),
            pltpu.VMEM((GE, FEAT), jnp.float32),
            pltpu.VMEM((GE, FEAT), jnp.float32),
            pltpu.VMEM_SHARED((ACC_ROWS, FEAT), jnp.float32),
            pltpu.SemaphoreType.DMA,
            pltpu.SemaphoreType.DMA,
            pltpu.SemaphoreType.DMA,
            pltpu.SemaphoreType.DMA,
        ],
        compiler_params=pltpu.CompilerParams(use_tc_tiling_on_sc=False),
    )
    return f(src_flat, dst_flat, p, zeros_f)


_PBLK = 1024


def _pgrid():
    return (-(-ACC_PACK // _PBLK),)


def _pspec():
    return pl.BlockSpec((_PBLK, 128), lambda i: (i, 0))


def _fspec(r, c):
    return pl.BlockSpec((r, c), lambda i: (0, 0))


def _stage_a_body(a0, a1, xp, wb, bmat, d_out, p_out):
    deg = jnp.dot(a0[...] + a1[...], bmat[...],
                  preferred_element_type=jnp.float32) + 1.0
    dd = lax.rsqrt(deg)
    d_out[...] = dd
    xw = jnp.dot(xp[...], wb[...], preferred_element_type=jnp.float32)
    p_out[...] = dd * xw


def _stage_mid_body(a0, a1, p, d, bt, wb, out):
    dd = d[...]
    h = dd * (a0[...] + a1[...] + p[...]) + bt[...]
    out[...] = dd * jnp.dot(jnp.tanh(h), wb[...],
                            preferred_element_type=jnp.float32)


def _stage_last_body(a0, a1, p, d, bt, out):
    out[...] = d[...] * (a0[...] + a1[...] + p[...]) + bt[...]


def _tc_stage_a(a0p, a1p, xp, w1b, bmat):
    return pl.pallas_call(
        _stage_a_body,
        grid=_pgrid(),
        in_specs=[_pspec(), _pspec(), _pspec(),
                  _fspec(128, 128), _fspec(128, 128)],
        out_specs=[_pspec(), _pspec()],
        out_shape=[jax.ShapeDtypeStruct((ACC_PACK, 128), jnp.float32),
                   jax.ShapeDtypeStruct((ACC_PACK, 128), jnp.float32)],
    )(a0p, a1p, xp, w1b, bmat)


def _tc_stage_mid(a0p, a1p, pp, dp, bt, wbig):
    return pl.pallas_call(
        _stage_mid_body,
        grid=_pgrid(),
        in_specs=[_pspec(), _pspec(), _pspec(), _pspec(),
                  _fspec(1, 128), _fspec(128, 128)],
        out_specs=_pspec(),
        out_shape=jax.ShapeDtypeStruct((ACC_PACK, 128), jnp.float32),
    )(a0p, a1p, pp, dp, bt, wbig)


def _tc_stage_last(a0p, a1p, pp, dp, bt):
    return pl.pallas_call(
        _stage_last_body,
        grid=_pgrid(),
        in_specs=[_pspec(), _pspec(), _pspec(), _pspec(), _fspec(1, 128)],
        out_specs=_pspec(),
        out_shape=jax.ShapeDtypeStruct((ACC_PACK, 128), jnp.float32),
    )(a0p, a1p, pp, dp, bt)


def _packed(a):
    return a.reshape(ACC_PACK, 128)


def kernel(x, edge_index, W1, b1, W2, b2, W3, b3):
    n = x.shape[0]
    e = edge_index.shape[1]
    gpt = _groups_per_tile(e)
    e_pad = gpt * NW * GE

    src = edge_index[0].astype(jnp.int32)
    dst = edge_index[1].astype(jnp.int32)
    pad = e_pad - e
    src_flat = jnp.concatenate([src, jnp.zeros((pad,), jnp.int32)])
    dst_flat = jnp.concatenate([dst, jnp.full((pad,), n, jnp.int32)])

    eye = jnp.eye(PACK, dtype=jnp.float32)

    def kr(w):
        return jnp.kron(eye, jnp.pad(w, ((0, FEAT - w.shape[0]),
                                         (0, FEAT - w.shape[1]))))

    w1b = kr(W1)
    w2b = kr(W2)
    w3b = kr(W3)
    bcast = jnp.kron(eye, jnp.zeros((FEAT, FEAT), jnp.float32)
                     .at[0, :].set(1.0))
    b1t = jnp.tile(jnp.pad(b1, (0, FEAT - b1.shape[0])), PACK).reshape(1, 128)
    b2t = jnp.tile(jnp.pad(b2, (0, FEAT - b2.shape[0])), PACK).reshape(1, 128)
    b3t = jnp.tile(jnp.pad(b3, (0, FEAT - b3.shape[0])), PACK).reshape(1, 128)
    xp = jnp.pad(x, ((0, ACC_ROWS - n), (0, FEAT - x.shape[1]))
                 ).reshape(ACC_PACK, 128)

    zeros_f = jnp.zeros((ACC_ROWS, FEAT), jnp.float32)
    ones_blk = jnp.zeros((GE, FEAT), jnp.float32).at[:, 0].set(1.0)

    deg0, deg1 = _sc_degree(dst_flat, zeros_f, ones_blk, gpt)
    dp, p1p = _tc_stage_a(_packed(deg0), _packed(deg1), xp, w1b, bcast)

    a0, a1 = _sc_aggregate(src_flat, dst_flat,
                           p1p.reshape(ACC_ROWS, FEAT), zeros_f, gpt)
    p2p = _tc_stage_mid(_packed(a0), _packed(a1), p1p, dp, b1t, w2b)

    a0, a1 = _sc_aggregate(src_flat, dst_flat,
                           p2p.reshape(ACC_ROWS, FEAT), zeros_f, gpt)
    p3p = _tc_stage_mid(_packed(a0), _packed(a1), p2p, dp, b2t, w3b)

    a0, a1 = _sc_aggregate(src_flat, dst_flat,
                           p3p.reshape(ACC_ROWS, FEAT), zeros_f, gpt)
    res = _tc_stage_last(_packed(a0), _packed(a1), p3p, dp, b3t)
    return res.reshape(ACC_ROWS, FEAT)[:n, : W3.shape[1]]
